# baseline (speedup 1.0000x reference)
import jax
import jax.numpy as jnp
from jax.experimental import pallas as pl
from jax.experimental.pallas import tpu as pltpu

B, F, D = 2, 2048, 768
E = 8
EPS = 1e-3
FT = 1024
NT = B * FT


def _moe_kernel(x_ref, wr_ref, br_ref, w0_ref, w1_ref, wo_ref,
                s0_ref, b0_ref, s1_ref, b1_ref, bo_ref,
                out_ref, wts_ref):
    e = pl.program_id(1)

    @pl.when(e == 0)
    def _router():
        x = x_ref[...]
        feat = (x[0] + x[1]) * 0.5
        logits = jnp.dot(feat, wr_ref[...],
                         preferred_element_type=jnp.float32) + br_ref[...]
        w = jax.nn.softmax(logits, axis=-1)
        wts_ref[...] = jnp.concatenate([w, w], axis=0)

    xb = x_ref[...].reshape(NT, D)
    s0 = s0_ref[pl.ds(e, 1), :]
    b0 = b0_ref[pl.ds(e, 1), :].astype(jnp.bfloat16)
    s1 = s1_ref[pl.ds(e, 1), :]
    b1 = b1_ref[pl.ds(e, 1), :].astype(jnp.bfloat16)
    bo = bo_ref[pl.ds(e, 1), :]

    w0b = (w0_ref[0] * s0).astype(jnp.bfloat16)
    w1b = (w1_ref[0] * s1).astype(jnp.bfloat16)
    wob = wo_ref[0].astype(jnp.bfloat16)

    h = jnp.dot(xb, w0b,
                preferred_element_type=jnp.float32).astype(jnp.bfloat16)
    h = jnp.maximum(h + b0, 0)
    h = jnp.dot(h, w1b,
                preferred_element_type=jnp.float32).astype(jnp.bfloat16)
    h = jnp.maximum(h + b1, 0)
    y = jnp.dot(h, wob, preferred_element_type=jnp.float32)

    lane = jax.lax.broadcasted_iota(jnp.int32, (1, E), 1)
    sel = (lane == e).astype(jnp.float32)
    wc = jnp.sum(wts_ref[...] * sel, axis=1, keepdims=True)

    contrib = ((y + bo) * wc).reshape(B, FT, D)

    @pl.when(e == 0)
    def _init():
        out_ref[...] = contrib

    @pl.when(e > 0)
    def _acc():
        out_ref[...] += contrib


@jax.jit
def kernel(inputs, Wr, br, W0, b0, g0, be0, W1, b1, g1, be1, Wo, bo):
    inv = 1.0 / jnp.sqrt(1.0 + EPS)
    s0 = g0 * inv
    b0p = b0 * s0 + be0
    s1 = g1 * inv
    b1p = b1 * s1 + be1

    full = lambda *shape: pl.BlockSpec(shape, lambda ft, e: (0,) * len(shape))
    per_e = pl.BlockSpec((1, D, D), lambda ft, e: (e, 0, 0))

    out = pl.pallas_call(
        _moe_kernel,
        grid=(F // FT, E),
        in_specs=[
            pl.BlockSpec((B, FT, D), lambda ft, e: (0, ft, 0)),
            full(D, E),
            full(1, E),
            per_e, per_e, per_e,
            full(E, D), full(E, D),
            full(E, D), full(E, D),
            full(E, D),
        ],
        out_specs=pl.BlockSpec((B, FT, D), lambda ft, e: (0, ft, 0)),
        out_shape=jax.ShapeDtypeStruct((B, F, D), jnp.float32),
        scratch_shapes=[
            pltpu.VMEM((NT, E), jnp.float32),
        ],
        compiler_params=pltpu.CompilerParams(
            dimension_semantics=("arbitrary", "arbitrary"),
            vmem_limit_bytes=100 * 1024 * 1024,
        ),
    )(inputs.astype(jnp.bfloat16), Wr.astype(jnp.bfloat16),
      br.reshape(1, E), W0, W1, Wo, s0, b0p, s1, b1p, bo)
    return out


# speedup vs baseline: 1.0890x; 1.0890x over previous
import jax
import jax.numpy as jnp
from jax.experimental import pallas as pl
from jax.experimental.pallas import tpu as pltpu

B, F, D = 2, 2048, 768
E = 8
EPS = 1e-3
FT = 1024
NT = B * FT


def _moe_kernel(x_ref, wr_ref, br_ref, w0_ref, w1_ref, wo_ref,
                s0_ref, b0_ref, s1_ref, b1_ref, bo_ref,
                out_ref, xbf_ref, wcol_ref):
    e = pl.program_id(1)

    @pl.when(e == 0)
    def _router():
        x = x_ref[...]
        xbf_ref[...] = x.reshape(NT, D).astype(jnp.bfloat16)
        feat = (x[0] + x[1]) * 0.5
        logits = jnp.dot(feat, wr_ref[...],
                         preferred_element_type=jnp.float32) + br_ref[...]
        w = jax.nn.softmax(logits, axis=-1)
        wts = jnp.concatenate([w, w], axis=0)
        for j in range(E):
            wcol_ref[j] = wts[:, j:j + 1].astype(jnp.bfloat16)
        bias0 = jnp.dot(wts.astype(jnp.bfloat16),
                        bo_ref[...].astype(jnp.bfloat16),
                        preferred_element_type=jnp.float32)
        out_ref[...] = bias0.reshape(B, FT, D)

    xb = xbf_ref[...]
    s0 = s0_ref[pl.ds(e, 1), :]
    b0 = b0_ref[pl.ds(e, 1), :].astype(jnp.bfloat16)
    s1 = s1_ref[pl.ds(e, 1), :]
    b1 = b1_ref[pl.ds(e, 1), :].astype(jnp.bfloat16)

    w0b = (w0_ref[0] * s0).astype(jnp.bfloat16)
    w1b = (w1_ref[0] * s1).astype(jnp.bfloat16)
    wob = wo_ref[0].astype(jnp.bfloat16)

    h = jnp.dot(xb, w0b,
                preferred_element_type=jnp.float32).astype(jnp.bfloat16)
    h = jnp.maximum(h + b0, 0)
    h = jnp.dot(h, w1b,
                preferred_element_type=jnp.float32).astype(jnp.bfloat16)
    h = jnp.maximum(h + b1, 0)
    h = h * wcol_ref[e]
    y = jnp.dot(h, wob, preferred_element_type=jnp.float32)

    out_ref[...] += y.reshape(B, FT, D)


@jax.jit
def kernel(inputs, Wr, br, W0, b0, g0, be0, W1, b1, g1, be1, Wo, bo):
    inv = 1.0 / jnp.sqrt(1.0 + EPS)
    s0 = g0 * inv
    b0p = b0 * s0 + be0
    s1 = g1 * inv
    b1p = b1 * s1 + be1

    full = lambda *shape: pl.BlockSpec(shape, lambda ft, e: (0,) * len(shape))
    per_e = pl.BlockSpec((1, D, D), lambda ft, e: (e, 0, 0))

    out = pl.pallas_call(
        _moe_kernel,
        grid=(F // FT, E),
        in_specs=[
            pl.BlockSpec((B, FT, D), lambda ft, e: (0, ft, 0)),
            full(D, E),
            full(1, E),
            per_e, per_e, per_e,
            full(E, D), full(E, D),
            full(E, D), full(E, D),
            full(E, D),
        ],
        out_specs=pl.BlockSpec((B, FT, D), lambda ft, e: (0, ft, 0)),
        out_shape=jax.ShapeDtypeStruct((B, F, D), jnp.float32),
        scratch_shapes=[
            pltpu.VMEM((NT, D), jnp.bfloat16),
            pltpu.VMEM((E, NT, 1), jnp.bfloat16),
        ],
        compiler_params=pltpu.CompilerParams(
            dimension_semantics=("arbitrary", "arbitrary"),
            vmem_limit_bytes=100 * 1024 * 1024,
        ),
    )(inputs, Wr, br.reshape(1, E), W0, W1, Wo, s0, b0p, s1, b1p, bo)
    return out
